# baseline (speedup 1.0000x reference)
import functools

import jax
import jax.numpy as jnp
from jax import lax
from jax.experimental import pallas as pl
from jax.experimental.pallas import tpu as pltpu
from jax.experimental.pallas import tpu_sc as plsc

N = 10000
E = 160000
F = 128
K = 3

NP = 10240
EP = 163840
CE = 128
NTILES = 16
NCORES = 2
ROWS_PER_TILE = NP // NTILES
EDGES_PER_WORKER = EP // (NTILES * NCORES)
CHUNKS_PER_WORKER = EDGES_PER_WORKER // CE
EDGES_PER_TILE_DEG = EP // NTILES
CHUNKS_PER_TILE_DEG = EDGES_PER_TILE_DEG // CE
ACC_ROWS = 10112
ACC_RPT = ACC_ROWS // NTILES
ACC_CHUNKS = (128, 128, 128, 128, 120)
CHUNKS_C0 = 32

@functools.lru_cache(maxsize=None)
def _mesh():
    return plsc.VectorSubcoreMesh(core_axis_name="c", subcore_axis_name="s",
                                  num_cores=NCORES, num_subcores=NTILES)


def _dyn_splat(v16, lane):
    idx = jnp.full((16, 1), lane, jnp.int32)
    dnums = lax.GatherDimensionNumbers(
        offset_dims=(), collapsed_slice_dims=(0,), start_index_map=(0,))
    return lax.gather(v16, idx, dnums, (1,),
                      mode=lax.GatherScatterMode.PROMISE_IN_BOUNDS)


def _deg_body(dst_hbm, ew_hbm, deg_hbm, idx_b, ew_b, z_b, sem, deg_sh):
    sid = lax.axis_index("s")
    cid = lax.axis_index("c")
    z16 = jnp.zeros((16,), jnp.float32)

    def zero_b(i, _):
        z_b[pl.ds(i * 16, 16)] = z16
        return _
    lax.fori_loop(0, CE // 16, zero_b, None)
    for q in range(ROWS_PER_TILE // CE):
        pltpu.sync_copy(z_b, deg_sh.at[pl.ds(sid * ROWS_PER_TILE + q * CE, CE)])
    plsc.subcore_barrier()

    def deg_chunk(i, _):
        base = sid * EDGES_PER_TILE_DEG + i * CE
        pltpu.sync_copy(dst_hbm.at[pl.ds(base, CE)], idx_b)
        pltpu.sync_copy(ew_hbm.at[pl.ds(base, CE)], ew_b)
        pltpu.sync_copy(ew_b, deg_sh.at[idx_b], add=True)
        return _
    lax.fori_loop(0, CHUNKS_PER_TILE_DEG, deg_chunk, None)
    plsc.subcore_barrier()

    pltpu.sync_copy(deg_sh.at[pl.ds(sid * ROWS_PER_TILE, ROWS_PER_TILE)],
                    deg_hbm.at[pl.ds(cid * NP + sid * ROWS_PER_TILE, ROWS_PER_TILE)])


@functools.lru_cache(maxsize=None)
def _deg_call():
    return pl.kernel(
        _deg_body,
        out_type=jax.ShapeDtypeStruct((NCORES * NP,), jnp.float32),
        mesh=_mesh(),
        scratch_types=[
            pltpu.VMEM((CE,), jnp.int32),
            pltpu.VMEM((CE,), jnp.float32),
            pltpu.VMEM((CE,), jnp.float32),
            pltpu.SemaphoreType.DMA,
            pltpu.VMEM_SHARED((NP,), jnp.float32),
        ],
    )


def _wn_body(src_hbm, dst_hbm, ew_hbm, d_hbm, wn_hbm,
             idx_b, ew_b, wn_b, a_b, b_b, sem):
    sid = lax.axis_index("s")
    cid = lax.axis_index("c")
    wid = sid * NCORES + cid
    off = (cid * NP).astype(jnp.int32)

    def wn_chunk(i, _):
        base = wid * EDGES_PER_WORKER + i * CE
        pltpu.sync_copy(src_hbm.at[pl.ds(base, CE)], idx_b)
        for j in range(CE // 16):
            idx_b[pl.ds(j * 16, 16)] = idx_b[pl.ds(j * 16, 16)] + off
        pltpu.async_copy(d_hbm.at[idx_b], a_b, sem).wait()
        pltpu.sync_copy(dst_hbm.at[pl.ds(base, CE)], idx_b)
        for j in range(CE // 16):
            idx_b[pl.ds(j * 16, 16)] = idx_b[pl.ds(j * 16, 16)] + off
        pltpu.async_copy(d_hbm.at[idx_b], b_b, sem).wait()
        pltpu.sync_copy(ew_hbm.at[pl.ds(base, CE)], ew_b)
        for j in range(CE // 16):
            wn_b[pl.ds(j * 16, 16)] = (ew_b[pl.ds(j * 16, 16)]
                                       * a_b[pl.ds(j * 16, 16)]
                                       * b_b[pl.ds(j * 16, 16)])
        pltpu.sync_copy(wn_b, wn_hbm.at[pl.ds(base, CE)])
        return _
    lax.fori_loop(0, CHUNKS_PER_WORKER, wn_chunk, None)


@functools.lru_cache(maxsize=None)
def _wn_call():
    return pl.kernel(
        _wn_body,
        out_type=jax.ShapeDtypeStruct((EP,), jnp.float32),
        mesh=_mesh(),
        scratch_types=[
            pltpu.VMEM((CE,), jnp.int32),
            pltpu.VMEM((CE,), jnp.float32),
            pltpu.VMEM((CE,), jnp.float32),
            pltpu.VMEM((CE,), jnp.float32),
            pltpu.VMEM((CE,), jnp.float32),
            pltpu.SemaphoreType.DMA,
        ],
    )


def _prop_body(nb, v_hbm, src_hbm, dst_hbm, wn_hbm, out_hbm,
               gidx0, gidx1, didx0, didx1, wn_b0, wn_b1, rows0, rows1,
               gsem0, gsem1, ssem0, ssem1, acc):
    sid = lax.axis_index("s")
    cid = lax.axis_index("c")
    z16 = jnp.zeros((16,), jnp.float32)
    pair_chunks = 2 * CHUNKS_PER_WORKER
    n_chunks = jnp.where(cid == 0, CHUNKS_C0, pair_chunks - CHUNKS_C0)
    coff = jnp.where(cid == 0, 0, CHUNKS_C0)
    gidx = (gidx0, gidx1)
    didx = (didx0, didx1)
    wn_b = (wn_b0, wn_b1)
    rows = (rows0, rows1)
    gsem = (gsem0, gsem1)
    ssem = (ssem0, ssem1)

    def zero_row(r, _):
        for j in range(F // 16):
            rows0[r, pl.ds(j * 16, 16)] = z16
        return _

    def issue_gather(b, i, p):
        base = (sid * pair_chunks + coff + i) * CE
        pltpu.sync_copy(src_hbm.at[pl.ds(base, CE)], gidx[p])
        if b > 0:
            for j in range(CE // 16):
                gidx[p][pl.ds(j * 16, 16)] = gidx[p][pl.ds(j * 16, 16)] + jnp.int32(b * NP)
        pltpu.sync_copy(dst_hbm.at[pl.ds(base, CE)], didx[p])
        pltpu.sync_copy(wn_hbm.at[pl.ds(base, CE)], wn_b[p])
        pltpu.async_copy(v_hbm.at[gidx[p]], rows[p], gsem[p])

    def scale_scatter(p):
        pltpu.make_async_copy(v_hbm.at[gidx[p]], rows[p], gsem[p]).wait()
        for j16 in range(CE // 16):
            wn16 = wn_b[p][pl.ds(j16 * 16, 16)]
            for e_in in range(16):
                spl = _dyn_splat(wn16, e_in)
                e = j16 * 16 + e_in
                for j in range(F // 16):
                    rows[p][e, pl.ds(j * 16, 16)] = rows[p][e, pl.ds(j * 16, 16)] * spl
        pltpu.async_copy(rows[p], acc.at[didx[p]], ssem[p], add=True)

    def drain_scatter(p):
        pltpu.make_async_copy(rows[p], acc.at[didx[p]], ssem[p]).wait()

    for b in range(nb):
        lax.fori_loop(0, CE, zero_row, None)
        off_q = 0
        for ch in ACC_CHUNKS:
            pltpu.sync_copy(rows0.at[pl.ds(0, ch)],
                            acc.at[pl.ds(sid * ACC_RPT + off_q, ch)])
            off_q += ch
        plsc.subcore_barrier()

        issue_gather(b, 0, 0)
        issue_gather(b, 1, 1)

        def chunk_pair(i2, _):
            i0 = 2 * i2

            def half(p):
                scale_scatter(p)

                @pl.when(i0 + p + 2 < n_chunks)
                def _prefetch():
                    drain_scatter(p)
                    issue_gather(b, i0 + p + 2, p)
            half(0)
            half(1)
            return _
        lax.fori_loop(0, n_chunks // 2, chunk_pair, None)
        drain_scatter(0)
        drain_scatter(1)

        plsc.subcore_barrier()
        row0 = (cid * nb + b) * NP + sid * ACC_RPT
        off_q = 0
        for ch in ACC_CHUNKS:
            pltpu.sync_copy(acc.at[pl.ds(sid * ACC_RPT + off_q, ch)],
                            out_hbm.at[pl.ds(row0 + off_q, ch)])
            off_q += ch
        if b + 1 < nb:
            plsc.subcore_barrier()


@functools.lru_cache(maxsize=None)
def _make_prop(nb):
    return pl.kernel(
        functools.partial(_prop_body, nb),
        out_type=jax.ShapeDtypeStruct((NCORES * nb * NP, F), jnp.float32),
        mesh=_mesh(),
        scratch_types=[
            pltpu.VMEM((CE,), jnp.int32),
            pltpu.VMEM((CE,), jnp.int32),
            pltpu.VMEM((CE,), jnp.int32),
            pltpu.VMEM((CE,), jnp.int32),
            pltpu.VMEM((CE,), jnp.float32),
            pltpu.VMEM((CE,), jnp.float32),
            pltpu.VMEM((CE, F), jnp.float32),
            pltpu.VMEM((CE, F), jnp.float32),
            pltpu.SemaphoreType.DMA,
            pltpu.SemaphoreType.DMA,
            pltpu.SemaphoreType.DMA,
            pltpu.SemaphoreType.DMA,
            pltpu.VMEM_SHARED((ACC_ROWS, F), jnp.float32),
        ],
    )


_BLK = 1000
_GRID = N // _BLK


def _row_spec(w):
    return pl.BlockSpec((_BLK, w), lambda i: (i, 0))


def _full_spec(shape):
    nd = len(shape)
    return pl.BlockSpec(shape, lambda i, _nd=nd: (0,) * _nd)


def _tcA_body(tx0, tx1, tx2, wz, wr, wc, bz, br, bc,
              gz0, gz1, gz2, gr0, gr1, gr2, gc0, gc1, gc2, h10, h11, h12):
    txs = (tx0[...], tx1[...], tx2[...])
    gz_o = (gz0, gz1, gz2)
    gr_o = (gr0, gr1, gr2)
    gc_o = (gc0, gc1, gc2)
    h1_o = (h10, h11, h12)
    for k in range(K):
        gz = jnp.dot(txs[k], wz[k], preferred_element_type=jnp.float32) + bz[k]
        gr = jnp.dot(txs[k], wr[k], preferred_element_type=jnp.float32) + br[k]
        gc = jnp.dot(txs[k], wc[k], preferred_element_type=jnp.float32) + bc[k]
        z1 = jax.nn.sigmoid(gz)
        c1 = jnp.tanh(gc)
        gz_o[k][...] = gz
        gr_o[k][...] = gr
        gc_o[k][...] = gc
        h1_o[k][...] = (1.0 - z1) * c1


def _tcB_body(th0, th1, th2, gr0, gr1, gr2, gz0, gz1, gz2, h10, h11, h12,
              whr, whz, z20, z21, z22, rh0, rh1, rh2):
    ths = (th0[...], th1[...], th2[...])
    grs = (gr0[...], gr1[...], gr2[...])
    gzs = (gz0[...], gz1[...], gz2[...])
    h1s = (h10[...], h11[...], h12[...])
    z2_o = (z20, z21, z22)
    rh_o = (rh0, rh1, rh2)
    for k in range(K):
        r2 = jax.nn.sigmoid(grs[k] + jnp.dot(ths[k], whr[k], preferred_element_type=jnp.float32))
        z2 = jax.nn.sigmoid(gzs[k] + jnp.dot(ths[k], whz[k], preferred_element_type=jnp.float32))
        z2_o[k][...] = z2
        rh_o[k][...] = r2 * h1s[k]


def _tcC_body(trh0, trh1, trh2, gc0, gc1, gc2, z20, z21, z22, h10, h11, h12,
              whc, wl, bl, out):
    trhs = (trh0[...], trh1[...], trh2[...])
    gcs = (gc0[...], gc1[...], gc2[...])
    z2s = (z20[...], z21[...], z22[...])
    h1s = (h10[...], h11[...], h12[...])
    o = jnp.zeros_like(gcs[0]) + bl[0, 0]
    for k in range(K):
        c2 = jnp.tanh(gcs[k] + jnp.dot(trhs[k], whc[k], preferred_element_type=jnp.float32))
        h2 = z2s[k] * h1s[k] + (1.0 - z2s[k]) * c2
        o = o + h2 * wl[0, k]
    out[...] = jnp.maximum(o, 0.0)


def _nf():
    return jax.ShapeDtypeStruct((N, F), jnp.float32)


def _tcR_body(deg, d):
    d[...] = lax.rsqrt(deg[...] + 1e-6)


_tcR = pl.pallas_call(
    _tcR_body,
    out_shape=jax.ShapeDtypeStruct((NCORES * NP // F, F), jnp.float32),
)


_tcA = pl.pallas_call(
    _tcA_body,
    grid=(_GRID,),
    in_specs=[_row_spec(F)] * 3 + [_full_spec((K, F, F))] * 3 + [_full_spec((K, F))] * 3,
    out_specs=[_row_spec(F)] * 12,
    out_shape=[_nf()] * 12,
)

_tcB = pl.pallas_call(
    _tcB_body,
    grid=(_GRID,),
    in_specs=[_row_spec(F * K)] * 3 + [_row_spec(F)] * 9 + [_full_spec((K, F * K, F))] * 2,
    out_specs=[_row_spec(F)] * 6,
    out_shape=[_nf()] * 6,
)

_tcC = pl.pallas_call(
    _tcC_body,
    grid=(_GRID,),
    in_specs=[_row_spec(F * K)] * 3 + [_row_spec(F)] * 9
    + [_full_spec((K, F * K, F)), _full_spec((1, K)), _full_spec((1, 1))],
    out_specs=_row_spec(F),
    out_shape=_nf(),
)


def _pad_rows(a):
    return jnp.pad(a, ((0, NP - N), (0, 0)))


def _prop_full(v_np, srcp, dstp, wnp):
    p = _make_prop(1)(v_np, srcp, dstp, wnp)
    return p[:NP] + p[NP:]


def _cheb_wide(flat_nf3, srcp, dstp, wnp):
    v = jnp.stack([flat_nf3[:, :F], flat_nf3[:, F:2 * F], flat_nf3[:, 2 * F:]], axis=0)
    v_p = jnp.pad(v, ((0, 0), (0, NP - N), (0, 0)))
    v_flat = v_p.reshape(K * NP, F)
    p1 = _make_prop(3)(v_flat, srcp, dstp, wnp).reshape(NCORES, K, NP, F)
    t1_b = p1[0] + p1[1]
    p2 = _make_prop(3)(t1_b.reshape(K * NP, F), srcp, dstp, wnp).reshape(NCORES, K, NP, F)
    t2_b = 2.0 * (p2[0] + p2[1]) - v_p
    t1 = jnp.concatenate([t1_b[0, :N], t1_b[1, :N], t1_b[2, :N]], axis=1)
    t2 = jnp.concatenate([t2_b[0, :N], t2_b[1, :N], t2_b[2, :N]], axis=1)
    return t1, t2


def kernel(x, edge_index, edge_weight, W_xz, W_hz, W_xr, W_hr, W_xc, W_hc,
           b_z, b_r, b_c, W_lin, b_lin):
    src = edge_index[0]
    dst = edge_index[1]
    pad_e = EP - E
    srcp = jnp.concatenate([src, jnp.zeros((pad_e,), jnp.int32)])
    dstp = jnp.concatenate([dst, jnp.zeros((pad_e,), jnp.int32)])
    ewp = jnp.concatenate([edge_weight, jnp.zeros((pad_e,), jnp.float32)])

    deg = _deg_call()(dstp, ewp)
    d = _tcR(deg.reshape(NCORES * NP // F, F)).reshape(NCORES * NP)
    wnp = _wn_call()(srcp, dstp, ewp, d)
    wn = wnp[:E]

    xp = _pad_rows(x)
    t1x = _prop_full(xp, srcp, dstp, wnp)
    t2x = 2.0 * _prop_full(t1x, srcp, dstp, wnp) - xp
    tx1 = t1x[:N]
    tx2 = t2x[:N]

    bzT = b_z.T
    brT = b_r.T
    bcT = b_c.T

    (gz0, gz1, gz2, gr0, gr1, gr2, gc0, gc1, gc2, h10, h11, h12) = _tcA(
        x, tx1, tx2, W_xz, W_xr, W_xc, bzT, brT, bcT)

    h1flat = jnp.stack([h10, h11, h12], axis=-1).reshape(N, F * K)
    th1, th2 = _cheb_wide(h1flat, srcp, dstp, wnp)

    (z20, z21, z22, rh0, rh1, rh2) = _tcB(
        h1flat, th1, th2, gr0, gr1, gr2, gz0, gz1, gz2, h10, h11, h12,
        W_hr, W_hz)

    rhflat = jnp.stack([rh0, rh1, rh2], axis=-1).reshape(N, F * K)
    trh1, trh2 = _cheb_wide(rhflat, srcp, dstp, wnp)

    out = _tcC(rhflat, trh1, trh2, gc0, gc1, gc2, z20, z21, z22, h10, h11, h12,
               W_hc, W_lin.reshape(1, K), b_lin.reshape(1, 1))

    return out.reshape(N, F, 1), wn


# speedup vs baseline: 1.1639x; 1.1639x over previous
import functools

import jax
import jax.numpy as jnp
from jax import lax
from jax.experimental import pallas as pl
from jax.experimental.pallas import tpu as pltpu
from jax.experimental.pallas import tpu_sc as plsc

N = 10000
E = 160000
F = 128
K = 3

NP = 10240
EP = 163840
CE = 128
NTILES = 16
NCORES = 2
ROWS_PER_TILE = NP // NTILES
EDGES_PER_WORKER = EP // (NTILES * NCORES)
CHUNKS_PER_WORKER = EDGES_PER_WORKER // CE
EDGES_PER_TILE_DEG = EP // NTILES
CHUNKS_PER_TILE_DEG = EDGES_PER_TILE_DEG // CE
ACC_ROWS = 10112
ACC_RPT = ACC_ROWS // NTILES
ACC_CHUNKS = (128, 128, 128, 128, 120)
CHUNKS_C0 = 48

@functools.lru_cache(maxsize=None)
def _mesh():
    return plsc.VectorSubcoreMesh(core_axis_name="c", subcore_axis_name="s",
                                  num_cores=NCORES, num_subcores=NTILES)


def _dyn_splat(v16, lane):
    idx = jnp.full((16, 1), lane, jnp.int32)
    dnums = lax.GatherDimensionNumbers(
        offset_dims=(), collapsed_slice_dims=(0,), start_index_map=(0,))
    return lax.gather(v16, idx, dnums, (1,),
                      mode=lax.GatherScatterMode.PROMISE_IN_BOUNDS)


def _deg_body(dst_hbm, ew_hbm, deg_hbm, idx_b, ew_b, z_b, sem, deg_sh):
    sid = lax.axis_index("s")
    cid = lax.axis_index("c")
    z16 = jnp.zeros((16,), jnp.float32)

    def zero_b(i, _):
        z_b[pl.ds(i * 16, 16)] = z16
        return _
    lax.fori_loop(0, CE // 16, zero_b, None)
    for q in range(ROWS_PER_TILE // CE):
        pltpu.sync_copy(z_b, deg_sh.at[pl.ds(sid * ROWS_PER_TILE + q * CE, CE)])
    plsc.subcore_barrier()

    def deg_chunk(i, _):
        base = sid * EDGES_PER_TILE_DEG + i * CE
        pltpu.sync_copy(dst_hbm.at[pl.ds(base, CE)], idx_b)
        pltpu.sync_copy(ew_hbm.at[pl.ds(base, CE)], ew_b)
        pltpu.sync_copy(ew_b, deg_sh.at[idx_b], add=True)
        return _
    lax.fori_loop(0, CHUNKS_PER_TILE_DEG, deg_chunk, None)
    plsc.subcore_barrier()

    pltpu.sync_copy(deg_sh.at[pl.ds(sid * ROWS_PER_TILE, ROWS_PER_TILE)],
                    deg_hbm.at[pl.ds(cid * NP + sid * ROWS_PER_TILE, ROWS_PER_TILE)])


@functools.lru_cache(maxsize=None)
def _deg_call():
    return pl.kernel(
        _deg_body,
        out_type=jax.ShapeDtypeStruct((NCORES * NP,), jnp.float32),
        mesh=_mesh(),
        scratch_types=[
            pltpu.VMEM((CE,), jnp.int32),
            pltpu.VMEM((CE,), jnp.float32),
            pltpu.VMEM((CE,), jnp.float32),
            pltpu.SemaphoreType.DMA,
            pltpu.VMEM_SHARED((NP,), jnp.float32),
        ],
    )


def _wn_body(src_hbm, dst_hbm, ew_hbm, d_hbm, wn_hbm,
             idx_b, ew_b, wn_b, a_b, b_b, sem):
    sid = lax.axis_index("s")
    cid = lax.axis_index("c")
    wid = sid * NCORES + cid
    off = (cid * NP).astype(jnp.int32)

    def wn_chunk(i, _):
        base = wid * EDGES_PER_WORKER + i * CE
        pltpu.sync_copy(src_hbm.at[pl.ds(base, CE)], idx_b)
        for j in range(CE // 16):
            idx_b[pl.ds(j * 16, 16)] = idx_b[pl.ds(j * 16, 16)] + off
        pltpu.async_copy(d_hbm.at[idx_b], a_b, sem).wait()
        pltpu.sync_copy(dst_hbm.at[pl.ds(base, CE)], idx_b)
        for j in range(CE // 16):
            idx_b[pl.ds(j * 16, 16)] = idx_b[pl.ds(j * 16, 16)] + off
        pltpu.async_copy(d_hbm.at[idx_b], b_b, sem).wait()
        pltpu.sync_copy(ew_hbm.at[pl.ds(base, CE)], ew_b)
        for j in range(CE // 16):
            wn_b[pl.ds(j * 16, 16)] = (ew_b[pl.ds(j * 16, 16)]
                                       * a_b[pl.ds(j * 16, 16)]
                                       * b_b[pl.ds(j * 16, 16)])
        pltpu.sync_copy(wn_b, wn_hbm.at[pl.ds(base, CE)])
        return _
    lax.fori_loop(0, CHUNKS_PER_WORKER, wn_chunk, None)


@functools.lru_cache(maxsize=None)
def _wn_call():
    return pl.kernel(
        _wn_body,
        out_type=jax.ShapeDtypeStruct((EP,), jnp.float32),
        mesh=_mesh(),
        scratch_types=[
            pltpu.VMEM((CE,), jnp.int32),
            pltpu.VMEM((CE,), jnp.float32),
            pltpu.VMEM((CE,), jnp.float32),
            pltpu.VMEM((CE,), jnp.float32),
            pltpu.VMEM((CE,), jnp.float32),
            pltpu.SemaphoreType.DMA,
        ],
    )


def _prop_body(nb, v_hbm, src_hbm, dst_hbm, wn_hbm, out_hbm,
               gidx0, gidx1, didx0, didx1, wn_b0, wn_b1, rows0, rows1,
               gsem0, gsem1, ssem0, ssem1, acc):
    sid = lax.axis_index("s")
    cid = lax.axis_index("c")
    z16 = jnp.zeros((16,), jnp.float32)
    pair_chunks = 2 * CHUNKS_PER_WORKER
    n_chunks = jnp.where(cid == 0, CHUNKS_C0, pair_chunks - CHUNKS_C0)
    coff = jnp.where(cid == 0, 0, CHUNKS_C0)
    gidx = (gidx0, gidx1)
    didx = (didx0, didx1)
    wn_b = (wn_b0, wn_b1)
    rows = (rows0, rows1)
    gsem = (gsem0, gsem1)
    ssem = (ssem0, ssem1)

    def zero_row(r, _):
        for j in range(F // 16):
            rows0[r, pl.ds(j * 16, 16)] = z16
        return _

    def issue_gather(b, i, p):
        base = (sid * pair_chunks + coff + i) * CE
        pltpu.sync_copy(src_hbm.at[pl.ds(base, CE)], gidx[p])
        if b > 0:
            for j in range(CE // 16):
                gidx[p][pl.ds(j * 16, 16)] = gidx[p][pl.ds(j * 16, 16)] + jnp.int32(b * NP)
        pltpu.sync_copy(dst_hbm.at[pl.ds(base, CE)], didx[p])
        pltpu.sync_copy(wn_hbm.at[pl.ds(base, CE)], wn_b[p])
        pltpu.async_copy(v_hbm.at[gidx[p]], rows[p], gsem[p])

    def scale_scatter(p):
        pltpu.make_async_copy(v_hbm.at[gidx[p]], rows[p], gsem[p]).wait()
        for j16 in range(CE // 16):
            wn16 = wn_b[p][pl.ds(j16 * 16, 16)]
            for e_in in range(16):
                spl = _dyn_splat(wn16, e_in)
                e = j16 * 16 + e_in
                for j in range(F // 16):
                    rows[p][e, pl.ds(j * 16, 16)] = rows[p][e, pl.ds(j * 16, 16)] * spl
        pltpu.async_copy(rows[p], acc.at[didx[p]], ssem[p], add=True)

    def drain_scatter(p):
        pltpu.make_async_copy(rows[p], acc.at[didx[p]], ssem[p]).wait()

    for b in range(nb):
        lax.fori_loop(0, CE, zero_row, None)
        off_q = 0
        for ch in ACC_CHUNKS:
            pltpu.sync_copy(rows0.at[pl.ds(0, ch)],
                            acc.at[pl.ds(sid * ACC_RPT + off_q, ch)])
            off_q += ch
        plsc.subcore_barrier()

        issue_gather(b, 0, 0)
        issue_gather(b, 1, 1)

        def chunk_pair(i2, _):
            i0 = 2 * i2

            def half(p):
                scale_scatter(p)

                @pl.when(i0 + p + 2 < n_chunks)
                def _prefetch():
                    drain_scatter(p)
                    issue_gather(b, i0 + p + 2, p)
            half(0)
            half(1)
            return _
        lax.fori_loop(0, n_chunks // 2, chunk_pair, None)
        drain_scatter(0)
        drain_scatter(1)

        plsc.subcore_barrier()
        row0 = (cid * nb + b) * NP + sid * ACC_RPT
        off_q = 0
        for ch in ACC_CHUNKS:
            pltpu.sync_copy(acc.at[pl.ds(sid * ACC_RPT + off_q, ch)],
                            out_hbm.at[pl.ds(row0 + off_q, ch)])
            off_q += ch
        if b + 1 < nb:
            plsc.subcore_barrier()


@functools.lru_cache(maxsize=None)
def _make_prop(nb):
    return pl.kernel(
        functools.partial(_prop_body, nb),
        out_type=jax.ShapeDtypeStruct((NCORES * nb * NP, F), jnp.float32),
        mesh=_mesh(),
        scratch_types=[
            pltpu.VMEM((CE,), jnp.int32),
            pltpu.VMEM((CE,), jnp.int32),
            pltpu.VMEM((CE,), jnp.int32),
            pltpu.VMEM((CE,), jnp.int32),
            pltpu.VMEM((CE,), jnp.float32),
            pltpu.VMEM((CE,), jnp.float32),
            pltpu.VMEM((CE, F), jnp.float32),
            pltpu.VMEM((CE, F), jnp.float32),
            pltpu.SemaphoreType.DMA,
            pltpu.SemaphoreType.DMA,
            pltpu.SemaphoreType.DMA,
            pltpu.SemaphoreType.DMA,
            pltpu.VMEM_SHARED((ACC_ROWS, F), jnp.float32),
        ],
    )


_BLK = 1000
_GRID = N // _BLK


def _row_spec(w):
    return pl.BlockSpec((_BLK, w), lambda i: (i, 0))


def _full_spec(shape):
    nd = len(shape)
    return pl.BlockSpec(shape, lambda i, _nd=nd: (0,) * _nd)


def _tcA_body(tx0, tx1, tx2, wz, wr, wc, bz, br, bc,
              gz0, gz1, gz2, gr0, gr1, gr2, gc0, gc1, gc2, h10, h11, h12):
    txs = (tx0[...], tx1[...], tx2[...])
    gz_o = (gz0, gz1, gz2)
    gr_o = (gr0, gr1, gr2)
    gc_o = (gc0, gc1, gc2)
    h1_o = (h10, h11, h12)
    for k in range(K):
        gz = jnp.dot(txs[k], wz[k], preferred_element_type=jnp.float32) + bz[k]
        gr = jnp.dot(txs[k], wr[k], preferred_element_type=jnp.float32) + br[k]
        gc = jnp.dot(txs[k], wc[k], preferred_element_type=jnp.float32) + bc[k]
        z1 = jax.nn.sigmoid(gz)
        c1 = jnp.tanh(gc)
        gz_o[k][...] = gz
        gr_o[k][...] = gr
        gc_o[k][...] = gc
        h1_o[k][...] = (1.0 - z1) * c1


def _tcB_body(th0, th1, th2, gr0, gr1, gr2, gz0, gz1, gz2, h10, h11, h12,
              whr, whz, z20, z21, z22, rh0, rh1, rh2):
    ths = (th0[...], th1[...], th2[...])
    grs = (gr0[...], gr1[...], gr2[...])
    gzs = (gz0[...], gz1[...], gz2[...])
    h1s = (h10[...], h11[...], h12[...])
    z2_o = (z20, z21, z22)
    rh_o = (rh0, rh1, rh2)
    for k in range(K):
        r2 = jax.nn.sigmoid(grs[k] + jnp.dot(ths[k], whr[k], preferred_element_type=jnp.float32))
        z2 = jax.nn.sigmoid(gzs[k] + jnp.dot(ths[k], whz[k], preferred_element_type=jnp.float32))
        z2_o[k][...] = z2
        rh_o[k][...] = r2 * h1s[k]


def _tcC_body(trh0, trh1, trh2, gc0, gc1, gc2, z20, z21, z22, h10, h11, h12,
              whc, wl, bl, out):
    trhs = (trh0[...], trh1[...], trh2[...])
    gcs = (gc0[...], gc1[...], gc2[...])
    z2s = (z20[...], z21[...], z22[...])
    h1s = (h10[...], h11[...], h12[...])
    o = jnp.zeros_like(gcs[0]) + bl[0, 0]
    for k in range(K):
        c2 = jnp.tanh(gcs[k] + jnp.dot(trhs[k], whc[k], preferred_element_type=jnp.float32))
        h2 = z2s[k] * h1s[k] + (1.0 - z2s[k]) * c2
        o = o + h2 * wl[0, k]
    out[...] = jnp.maximum(o, 0.0)


def _nf():
    return jax.ShapeDtypeStruct((N, F), jnp.float32)


def _tcR_body(deg, d):
    d[...] = lax.rsqrt(deg[...] + 1e-6)


_tcR = pl.pallas_call(
    _tcR_body,
    out_shape=jax.ShapeDtypeStruct((NCORES * NP // F, F), jnp.float32),
)


_tcA = pl.pallas_call(
    _tcA_body,
    grid=(_GRID,),
    in_specs=[_row_spec(F)] * 3 + [_full_spec((K, F, F))] * 3 + [_full_spec((K, F))] * 3,
    out_specs=[_row_spec(F)] * 12,
    out_shape=[_nf()] * 12,
)

_tcB = pl.pallas_call(
    _tcB_body,
    grid=(_GRID,),
    in_specs=[_row_spec(F * K)] * 3 + [_row_spec(F)] * 9 + [_full_spec((K, F * K, F))] * 2,
    out_specs=[_row_spec(F)] * 6,
    out_shape=[_nf()] * 6,
)

_tcC = pl.pallas_call(
    _tcC_body,
    grid=(_GRID,),
    in_specs=[_row_spec(F * K)] * 3 + [_row_spec(F)] * 9
    + [_full_spec((K, F * K, F)), _full_spec((1, K)), _full_spec((1, 1))],
    out_specs=_row_spec(F),
    out_shape=_nf(),
)


def _pad_rows(a):
    return jnp.pad(a, ((0, NP - N), (0, 0)))


def _prop_full(v_np, srcp, dstp, wnp):
    p = _make_prop(1)(v_np, srcp, dstp, wnp)
    return p[:NP] + p[NP:]


def _cheb_wide(flat_nf3, srcp, dstp, wnp):
    v = jnp.stack([flat_nf3[:, :F], flat_nf3[:, F:2 * F], flat_nf3[:, 2 * F:]], axis=0)
    v_p = jnp.pad(v, ((0, 0), (0, NP - N), (0, 0)))
    v_flat = v_p.reshape(K * NP, F)
    p1 = _make_prop(3)(v_flat, srcp, dstp, wnp).reshape(NCORES, K, NP, F)
    t1_b = p1[0] + p1[1]
    p2 = _make_prop(3)(t1_b.reshape(K * NP, F), srcp, dstp, wnp).reshape(NCORES, K, NP, F)
    t2_b = 2.0 * (p2[0] + p2[1]) - v_p
    t1 = jnp.concatenate([t1_b[0, :N], t1_b[1, :N], t1_b[2, :N]], axis=1)
    t2 = jnp.concatenate([t2_b[0, :N], t2_b[1, :N], t2_b[2, :N]], axis=1)
    return t1, t2


def kernel(x, edge_index, edge_weight, W_xz, W_hz, W_xr, W_hr, W_xc, W_hc,
           b_z, b_r, b_c, W_lin, b_lin):
    src = edge_index[0]
    dst = edge_index[1]
    pad_e = EP - E
    srcp = jnp.concatenate([src, jnp.zeros((pad_e,), jnp.int32)])
    dstp = jnp.concatenate([dst, jnp.zeros((pad_e,), jnp.int32)])
    ewp = jnp.concatenate([edge_weight, jnp.zeros((pad_e,), jnp.float32)])

    deg = _deg_call()(dstp, ewp)
    d = _tcR(deg.reshape(NCORES * NP // F, F)).reshape(NCORES * NP)
    wnp = _wn_call()(srcp, dstp, ewp, d)
    wn = wnp[:E]

    xp = _pad_rows(x)
    t1x = _prop_full(xp, srcp, dstp, wnp)
    t2x = 2.0 * _prop_full(t1x, srcp, dstp, wnp) - xp
    tx1 = t1x[:N]
    tx2 = t2x[:N]

    bzT = b_z.T
    brT = b_r.T
    bcT = b_c.T

    (gz0, gz1, gz2, gr0, gr1, gr2, gc0, gc1, gc2, h10, h11, h12) = _tcA(
        x, tx1, tx2, W_xz, W_xr, W_xc, bzT, brT, bcT)

    h1flat = jnp.stack([h10, h11, h12], axis=-1).reshape(N, F * K)
    th1, th2 = _cheb_wide(h1flat, srcp, dstp, wnp)

    (z20, z21, z22, rh0, rh1, rh2) = _tcB(
        h1flat, th1, th2, gr0, gr1, gr2, gz0, gz1, gz2, h10, h11, h12,
        W_hr, W_hz)

    rhflat = jnp.stack([rh0, rh1, rh2], axis=-1).reshape(N, F * K)
    trh1, trh2 = _cheb_wide(rhflat, srcp, dstp, wnp)

    out = _tcC(rhflat, trh1, trh2, gc0, gc1, gc2, z20, z21, z22, h10, h11, h12,
               W_hc, W_lin.reshape(1, K), b_lin.reshape(1, 1))

    return out.reshape(N, F, 1), wn
